# baseline (speedup 1.0000x reference)
import functools

import jax
import jax.numpy as jnp
from jax import lax
from jax.experimental import pallas as pl
from jax.experimental.pallas import tpu as pltpu
from jax.experimental.pallas import tpu_sc as plsc

NC, NS, L = 2, 16, 16
NW = NC * NS
CHUNK = 256


def kernel(idx, emb_weight):
    B = idx.shape[0]
    V, D = emb_weight.shape
    b_per_w = B // NW
    table2 = emb_weight.reshape(V // 2, 2 * D)

    mesh = plsc.VectorSubcoreMesh(core_axis_name="c", subcore_axis_name="s")

    @functools.partial(
        pl.kernel,
        mesh=mesh,
        out_type=jax.ShapeDtypeStruct((B, D), emb_weight.dtype),
        scratch_types=[
            pltpu.VMEM((b_per_w,), jnp.int32),
            pltpu.VMEM((CHUNK,), jnp.int32),
            pltpu.VMEM((CHUNK, 2 * D), emb_weight.dtype),
            pltpu.VMEM((CHUNK, D), emb_weight.dtype),
            pltpu.SemaphoreType.DMA,
        ],
        compiler_params=pltpu.CompilerParams(needs_layout_passes=False),
    )
    def gather_kernel(table_hbm, idx_hbm, out_hbm, idx_v, j_v, rows_v,
                      outb_v, sem):
        wid = lax.axis_index("s") * NC + lax.axis_index("c")
        base = wid * b_per_w
        pltpu.sync_copy(idx_hbm.at[pl.ds(base, b_per_w)], idx_v)

        @pl.loop(0, b_per_w, step=CHUNK)
        def _(i0):
            @pl.loop(0, CHUNK, step=L)
            def _(i):
                j_v[pl.ds(i, L)] = lax.shift_right_logical(
                    idx_v[pl.ds(i0 + i, L)], 1)

            pltpu.async_copy(table_hbm.at[j_v], rows_v, sem).wait()

            @pl.loop(0, CHUNK, step=L)
            def _(i):
                rvec = lax.iota(jnp.int32, L) + i
                pvec = (idx_v[pl.ds(i0 + i, L)] & 1) * D

                @pl.loop(0, D)
                def _(c):
                    cvec = lax.broadcast(c, (L,))
                    vals = plsc.load_gather(rows_v, [rvec, pvec + cvec])
                    plsc.store_scatter(outb_v, [rvec, cvec], vals)

            pltpu.sync_copy(outb_v, out_hbm.at[pl.ds(base + i0, CHUNK)])

    return gather_kernel(table2, idx.astype(jnp.int32))


# speedup vs baseline: 1.9172x; 1.9172x over previous
import functools

import jax
import jax.numpy as jnp
from jax import lax
from jax.experimental import pallas as pl
from jax.experimental.pallas import tpu as pltpu
from jax.experimental.pallas import tpu_sc as plsc

NC, NS, L = 2, 16, 16
NW = NC * NS
LAG = 4


def kernel(idx, emb_weight):
    B = idx.shape[0]
    V, D = emb_weight.shape
    b_per_w = B // NW

    mesh = plsc.VectorSubcoreMesh(core_axis_name="c", subcore_axis_name="s")

    @functools.partial(
        pl.kernel,
        mesh=mesh,
        out_type=jax.ShapeDtypeStruct((B, D), emb_weight.dtype),
        scratch_types=[
            pltpu.VMEM((b_per_w,), jnp.int32),
            pltpu.VMEM((b_per_w, D), emb_weight.dtype),
            pltpu.VMEM((L, D), emb_weight.dtype),
            pltpu.SemaphoreType.DMA,
        ],
        compiler_params=pltpu.CompilerParams(needs_layout_passes=False),
    )
    def gather_kernel(table_hbm, idx_hbm, out_hbm, idx_v, rows_v, drain_v,
                      sem):
        wid = lax.axis_index("s") * NC + lax.axis_index("c")
        base = wid * b_per_w
        pltpu.sync_copy(idx_hbm.at[pl.ds(base, b_per_w)], idx_v)
        lane = lax.iota(jnp.int32, L)

        @pl.loop(0, b_per_w, step=L)
        def _(i):
            v = idx_v[pl.ds(i, L)]
            for k in range(L):
                j = lax.reduce_sum(jnp.where(lane == k, v, 0), axes=(0,))
                pltpu.async_copy(
                    table_hbm.at[pl.ds(j, 1)],
                    rows_v.at[pl.ds(i + k, 1)],
                    sem,
                )

            @pl.when(i >= LAG * L)
            def _():
                pltpu.make_async_copy(
                    table_hbm.at[pl.ds(0, L)], drain_v, sem).wait()

        for _ in range(LAG):
            pltpu.make_async_copy(
                table_hbm.at[pl.ds(0, L)], drain_v, sem).wait()

        pltpu.sync_copy(rows_v, out_hbm.at[pl.ds(base, b_per_w)])

    return gather_kernel(emb_weight, idx.astype(jnp.int32))


# speedup vs baseline: 1.9636x; 1.0242x over previous
import functools

import jax
import jax.numpy as jnp
from jax import lax
from jax.experimental import pallas as pl
from jax.experimental.pallas import tpu as pltpu
from jax.experimental.pallas import tpu_sc as plsc

NC, NS, L = 2, 16, 16
NW = NC * NS
LAG = 8


def kernel(idx, emb_weight):
    B = idx.shape[0]
    V, D = emb_weight.shape
    b_per_w = B // NW

    mesh = plsc.VectorSubcoreMesh(core_axis_name="c", subcore_axis_name="s")

    @functools.partial(
        pl.kernel,
        mesh=mesh,
        out_type=jax.ShapeDtypeStruct((B, D), emb_weight.dtype),
        scratch_types=[
            pltpu.VMEM((b_per_w,), jnp.int32),
            pltpu.VMEM((b_per_w, D), emb_weight.dtype),
            pltpu.VMEM((L, D), emb_weight.dtype),
            pltpu.SemaphoreType.DMA,
        ],
        compiler_params=pltpu.CompilerParams(needs_layout_passes=False),
    )
    def gather_kernel(table_hbm, idx_hbm, out_hbm, idx_v, rows_v, drain_v,
                      sem):
        wid = lax.axis_index("s") * NC + lax.axis_index("c")
        base = wid * b_per_w
        pltpu.sync_copy(idx_hbm.at[pl.ds(base, b_per_w)], idx_v)
        lane = lax.iota(jnp.int32, L)

        @pl.loop(0, b_per_w, step=L)
        def _(i):
            v = idx_v[pl.ds(i, L)]
            for k in range(L):
                j = lax.reduce_sum(jnp.where(lane == k, v, 0), axes=(0,))
                pltpu.async_copy(
                    table_hbm.at[pl.ds(j, 1)],
                    rows_v.at[pl.ds(i + k, 1)],
                    sem,
                )

            @pl.when(i >= LAG * L)
            def _():
                pltpu.make_async_copy(
                    table_hbm.at[pl.ds(0, L)], drain_v, sem).wait()

        for _ in range(LAG):
            pltpu.make_async_copy(
                table_hbm.at[pl.ds(0, L)], drain_v, sem).wait()

        pltpu.sync_copy(rows_v, out_hbm.at[pl.ds(base, b_per_w)])

    return gather_kernel(emb_weight, idx.astype(jnp.int32))


# speedup vs baseline: 1.9840x; 1.0104x over previous
import functools

import jax
import jax.numpy as jnp
from jax import lax
from jax.experimental import pallas as pl
from jax.experimental.pallas import tpu as pltpu
from jax.experimental.pallas import tpu_sc as plsc

NC, NS, L = 2, 16, 16
NW = NC * NS
LAG = 12


def kernel(idx, emb_weight):
    B = idx.shape[0]
    V, D = emb_weight.shape
    b_per_w = B // NW

    mesh = plsc.VectorSubcoreMesh(core_axis_name="c", subcore_axis_name="s")

    @functools.partial(
        pl.kernel,
        mesh=mesh,
        out_type=jax.ShapeDtypeStruct((B, D), emb_weight.dtype),
        scratch_types=[
            pltpu.VMEM((b_per_w,), jnp.int32),
            pltpu.VMEM((b_per_w, D), emb_weight.dtype),
            pltpu.VMEM((L, D), emb_weight.dtype),
            pltpu.SemaphoreType.DMA,
        ],
        compiler_params=pltpu.CompilerParams(needs_layout_passes=False),
    )
    def gather_kernel(table_hbm, idx_hbm, out_hbm, idx_v, rows_v, drain_v,
                      sem):
        wid = lax.axis_index("s") * NC + lax.axis_index("c")
        base = wid * b_per_w
        pltpu.sync_copy(idx_hbm.at[pl.ds(base, b_per_w)], idx_v)
        lane = lax.iota(jnp.int32, L)

        @pl.loop(0, b_per_w, step=L)
        def _(i):
            v = idx_v[pl.ds(i, L)]
            for k in range(L):
                j = lax.reduce_sum(jnp.where(lane == k, v, 0), axes=(0,))
                pltpu.async_copy(
                    table_hbm.at[pl.ds(j, 1)],
                    rows_v.at[pl.ds(i + k, 1)],
                    sem,
                )

            @pl.when(i >= LAG * L)
            def _():
                pltpu.make_async_copy(
                    table_hbm.at[pl.ds(0, L)], drain_v, sem).wait()

        for _ in range(LAG):
            pltpu.make_async_copy(
                table_hbm.at[pl.ds(0, L)], drain_v, sem).wait()

        pltpu.sync_copy(rows_v, out_hbm.at[pl.ds(base, b_per_w)])

    return gather_kernel(emb_weight, idx.astype(jnp.int32))
